# baseline (speedup 1.0000x reference)
import functools

import jax
import jax.numpy as jnp
from jax import lax
from jax.experimental import pallas as pl
from jax.experimental.pallas import tpu as pltpu
from jax.experimental.pallas import tpu_sc as plsc

_C = 128
_LANES = 16


@functools.partial(jax.jit, static_argnames=("n_workers", "b", "l"))
def _run(idx2d, partidx, weight_groups, n_workers, b, l):
    n_rows, c = idx2d.shape
    n = n_rows * c
    p, d = weight_groups.shape
    dq_n = d // 8
    bq_n = b // _C
    per_w = n // n_workers
    chunks_w = per_w // _C
    bq_w = bq_n // n_workers
    groups = _C // _LANES
    steps = bq_w * l

    mesh = plsc.VectorSubcoreMesh(core_axis_name="c", subcore_axis_name="s")

    @functools.partial(
        pl.kernel,
        mesh=mesh,
        compiler_params=pltpu.CompilerParams(use_tc_tiling_on_sc=False,
                                             needs_layout_passes=False),
        out_type=jax.ShapeDtypeStruct((l, dq_n, bq_n, 8, _C), jnp.float32),
        scratch_types=[
            pltpu.VMEM((chunks_w, _C), jnp.int32),
            pltpu.VMEM((per_w,), jnp.int32),
            pltpu.VMEM((p, d), jnp.float32),
            pltpu.VMEM((2, 8, 8, _C), jnp.float32),
            pltpu.SemaphoreType.DMA,
            pltpu.SemaphoreType.DMA,
            pltpu.SemaphoreType.DMA,
            pltpu.SemaphoreType.DMA,
        ],
    )
    def sc_kernel(idx_hbm, part_hbm, wg_hbm, out_hbm,
                  idx_v, p_v, wg_v, tiles_v, semP, semW, semT0, semT1):
        wid = lax.axis_index("s") * 2 + lax.axis_index("c")
        row_base = wid * chunks_w

        pltpu.async_copy(wg_hbm, wg_v, semW)
        pltpu.sync_copy(idx_hbm.at[pl.ds(row_base, chunks_w)], idx_v)

        def fire_p(j, carry):
            pltpu.async_copy(part_hbm.at[idx_v.at[j]],
                             p_v.at[pl.ds(j * _C, _C)], semP)
            return carry
        lax.fori_loop(0, chunks_w, fire_p, 0)
        pltpu.make_async_copy(wg_hbm, wg_v, semW).wait()
        pltpu.make_async_copy(idx_hbm.at[pl.ds(row_base, chunks_w)], idx_v,
                              semP).wait()

        lane_l = lax.broadcasted_iota(jnp.int32, (_LANES,), 0) * l

        def drain_slot(s, sem):
            for dq in range(dq_n):
                pltpu.make_async_copy(tiles_v.at[s, dq], out_hbm.at[0, dq, 0],
                                      sem).wait()

        def step(i, s, sem):
            bq_local = i // l
            li = i % l
            bq = wid * bq_w + bq_local

            pg = []
            for g in range(groups):
                base = (bq_local * _C + g * _LANES) * l + li
                pg.append(plsc.load_gather(p_v, [lane_l + base]))

            def tile_dq(dq, carry2):
                for di in range(8):
                    dcol = jnp.full((_LANES,), dq * 8 + di, jnp.int32)
                    for g in range(groups):
                        vals = plsc.load_gather(wg_v, [pg[g], dcol])
                        tiles_v[s, dq, di, pl.ds(g * _LANES, _LANES)] = vals
                return carry2
            lax.fori_loop(0, dq_n, tile_dq, 0)

            for dq in range(dq_n):
                pltpu.async_copy(tiles_v.at[s, dq], out_hbm.at[li, dq, bq],
                                 sem)

        def pair(t, carry):
            @pl.when(t > 0)
            def _():
                drain_slot(0, semT0)
            step(2 * t, 0, semT0)

            @pl.when(t > 0)
            def _():
                drain_slot(1, semT1)
            step(2 * t + 1, 1, semT1)
            return carry

        lax.fori_loop(0, steps // 2, pair, 0)
        drain_slot(0, semT0)
        drain_slot(1, semT1)

    return sc_kernel(idx2d, partidx, weight_groups)


def kernel(input_idx, decoder, partidx, weight_groups):
    b, l = input_idx.shape
    d = weight_groups.shape[1]
    n = b * l
    idx2d = input_idx.reshape(n // _C, _C)
    out5d = _run(idx2d, partidx, weight_groups, 32, b, l)
    return out5d.transpose(2, 4, 0, 1, 3).reshape(b, l, d)


# speedup vs baseline: 1.6950x; 1.6950x over previous
import functools

import jax
import jax.numpy as jnp
from jax import lax
from jax.experimental import pallas as pl
from jax.experimental.pallas import tpu as pltpu
from jax.experimental.pallas import tpu_sc as plsc

_C = 128
_LANES = 16


@functools.partial(jax.jit, static_argnames=("n_workers", "b", "l"))
def _run(idx2d, partidx, weight_groups, n_workers, b, l):
    n_rows, c = idx2d.shape
    n = n_rows * c
    p, d = weight_groups.shape
    dq_n = d // 8
    bq_n = b // _C
    per_w = n // n_workers
    chunks_w = per_w // _C
    bq_w = bq_n // n_workers
    groups = _C // _LANES
    steps = bq_w * l

    mesh = plsc.VectorSubcoreMesh(core_axis_name="c", subcore_axis_name="s")

    @functools.partial(
        pl.kernel,
        mesh=mesh,
        compiler_params=pltpu.CompilerParams(use_tc_tiling_on_sc=False,
                                             needs_layout_passes=False),
        out_type=jax.ShapeDtypeStruct((l, dq_n, bq_n, 8, _C), jnp.float32),
        scratch_types=[
            pltpu.VMEM((chunks_w, _C), jnp.int32),
            pltpu.VMEM((per_w,), jnp.int32),
            pltpu.VMEM((p, d), jnp.float32),
            pltpu.VMEM((2, 64, _C), jnp.float32),
            pltpu.SemaphoreType.DMA,
            pltpu.SemaphoreType.DMA,
            pltpu.SemaphoreType.DMA,
            pltpu.SemaphoreType.DMA,
        ],
    )
    def sc_kernel(idx_hbm, part_hbm, wg_hbm, out_hbm,
                  idx_v, p_v, wg_v, tiles_v, semP, semW, semT0, semT1):
        wid = lax.axis_index("s") * 2 + lax.axis_index("c")
        row_base = wid * chunks_w

        pltpu.async_copy(wg_hbm, wg_v, semW)
        pltpu.sync_copy(idx_hbm.at[pl.ds(row_base, chunks_w)], idx_v)

        def fire_p(j, carry):
            pltpu.async_copy(part_hbm.at[idx_v.at[j]],
                             p_v.at[pl.ds(j * _C, _C)], semP)
            return carry
        lax.fori_loop(0, chunks_w, fire_p, 0)
        pltpu.make_async_copy(wg_hbm, wg_v, semW).wait()
        pltpu.make_async_copy(idx_hbm.at[pl.ds(row_base, chunks_w)], idx_v,
                              semP).wait()

        lane_l = lax.broadcasted_iota(jnp.int32, (_LANES,), 0) * l

        def drain_slot(s, sem):
            for dq in range(dq_n):
                pltpu.make_async_copy(tiles_v.at[s, pl.ds(dq * 8, 8)],
                                      out_hbm.at[0, dq, 0], sem).wait()

        def step(i, s, sem):
            bq_local = i // l
            li = i % l
            bq = wid * bq_w + bq_local

            pg = []
            for g in range(groups):
                base = (bq_local * _C + g * _LANES) * l + li
                pg.append(plsc.load_gather(p_v, [lane_l + base]))

            @plsc.parallel_loop(0, d, 1, unroll=8)
            def dloop(dv):
                dcol = jnp.full((_LANES,), dv, jnp.int32)
                for g in range(groups):
                    vals = plsc.load_gather(wg_v, [pg[g], dcol])
                    tiles_v[s, dv, pl.ds(g * _LANES, _LANES)] = vals

            for dq in range(dq_n):
                pltpu.async_copy(tiles_v.at[s, pl.ds(dq * 8, 8)],
                                 out_hbm.at[li, dq, bq], sem)

        def pair(t, carry):
            @pl.when(t > 0)
            def _():
                drain_slot(0, semT0)
            step(2 * t, 0, semT0)

            @pl.when(t > 0)
            def _():
                drain_slot(1, semT1)
            step(2 * t + 1, 1, semT1)
            return carry

        lax.fori_loop(0, steps // 2, pair, 0)
        drain_slot(0, semT0)
        drain_slot(1, semT1)

    return sc_kernel(idx2d, partidx, weight_groups)


def kernel(input_idx, decoder, partidx, weight_groups):
    b, l = input_idx.shape
    d = weight_groups.shape[1]
    n = b * l
    idx2d = input_idx.reshape(n // _C, _C)
    out5d = _run(idx2d, partidx, weight_groups, 32, b, l)
    return out5d.transpose(2, 4, 0, 1, 3).reshape(b, l, d)


# speedup vs baseline: 1.7170x; 1.0129x over previous
import functools

import jax
import jax.numpy as jnp
from jax import lax
from jax.experimental import pallas as pl
from jax.experimental.pallas import tpu as pltpu
from jax.experimental.pallas import tpu_sc as plsc

_C = 128
_LANES = 16


@functools.partial(jax.jit, static_argnames=("n_workers", "b", "l"))
def _run(idx2d, partidx, weight_groups, n_workers, b, l):
    n_rows, c = idx2d.shape
    n = n_rows * c
    p, d = weight_groups.shape
    dq_n = d // 8
    bq_n = b // _C
    per_w = n // n_workers
    chunks_w = per_w // _C
    bq_w = bq_n // n_workers
    groups = _C // _LANES
    steps = bq_w * l

    mesh = plsc.VectorSubcoreMesh(core_axis_name="c", subcore_axis_name="s")

    @functools.partial(
        pl.kernel,
        mesh=mesh,
        compiler_params=pltpu.CompilerParams(use_tc_tiling_on_sc=False,
                                             needs_layout_passes=False),
        out_type=jax.ShapeDtypeStruct((l, dq_n, bq_n, 8, _C), jnp.float32),
        scratch_types=[
            pltpu.VMEM((chunks_w, _C), jnp.int32),
            pltpu.VMEM((per_w,), jnp.int32),
            pltpu.VMEM((p, d), jnp.float32),
            pltpu.VMEM((2, 64, _C), jnp.float32),
            pltpu.SemaphoreType.DMA,
            pltpu.SemaphoreType.DMA,
            pltpu.SemaphoreType.DMA,
            pltpu.SemaphoreType.DMA,
        ],
    )
    def sc_kernel(idx_hbm, part_hbm, wg_hbm, out_hbm,
                  idx_v, p_v, wg_v, tiles_v, semP, semW, semT0, semT1):
        wid = lax.axis_index("s") * 2 + lax.axis_index("c")
        row_base = wid * chunks_w

        pltpu.async_copy(wg_hbm, wg_v, semW)
        pltpu.sync_copy(idx_hbm.at[pl.ds(row_base, chunks_w)], idx_v)

        def fire_p(j, carry):
            pltpu.async_copy(part_hbm.at[idx_v.at[j]],
                             p_v.at[pl.ds(j * _C, _C)], semP)
            return carry
        lax.fori_loop(0, chunks_w, fire_p, 0)
        pltpu.make_async_copy(wg_hbm, wg_v, semW).wait()
        pltpu.make_async_copy(idx_hbm.at[pl.ds(row_base, chunks_w)], idx_v,
                              semP).wait()

        lane_l = lax.broadcasted_iota(jnp.int32, (_LANES,), 0) * l

        def drain_slot(s, sem):
            for dq in range(dq_n):
                pltpu.make_async_copy(tiles_v.at[s, pl.ds(dq * 8, 8)],
                                      out_hbm.at[0, dq, 0], sem).wait()

        def step(i, s, sem):
            bq_local = i // l
            li = i % l
            bq = wid * bq_w + bq_local

            pg = []
            for g in range(groups):
                base = (bq_local * _C + g * _LANES) * l + li
                pg.append(plsc.load_gather(p_v, [lane_l + base]))

            dcol0 = jnp.zeros((_LANES,), jnp.int32)

            @plsc.parallel_loop(0, d, 1, unroll=16, carry=dcol0)
            def dloop(dv, dcol):
                for g in range(groups):
                    vals = plsc.load_gather(wg_v, [pg[g], dcol])
                    tiles_v[s, dv, pl.ds(g * _LANES, _LANES)] = vals
                return dcol + 1

            for dq in range(dq_n):
                pltpu.async_copy(tiles_v.at[s, pl.ds(dq * 8, 8)],
                                 out_hbm.at[li, dq, bq], sem)

        def pair(t, carry):
            @pl.when(t > 0)
            def _():
                drain_slot(0, semT0)
            step(2 * t, 0, semT0)

            @pl.when(t > 0)
            def _():
                drain_slot(1, semT1)
            step(2 * t + 1, 1, semT1)
            return carry

        lax.fori_loop(0, steps // 2, pair, 0)
        drain_slot(0, semT0)
        drain_slot(1, semT1)

    return sc_kernel(idx2d, partidx, weight_groups)


def kernel(input_idx, decoder, partidx, weight_groups):
    b, l = input_idx.shape
    d = weight_groups.shape[1]
    n = b * l
    idx2d = input_idx.reshape(n // _C, _C)
    out5d = _run(idx2d, partidx, weight_groups, 32, b, l)
    return out5d.transpose(2, 4, 0, 1, 3).reshape(b, l, d)


# speedup vs baseline: 1.7610x; 1.0256x over previous
import functools

import jax
import jax.numpy as jnp
from jax import lax
from jax.experimental import pallas as pl
from jax.experimental.pallas import tpu as pltpu
from jax.experimental.pallas import tpu_sc as plsc

_C = 128
_LANES = 16


@functools.partial(jax.jit, static_argnames=("n_workers", "b", "l"))
def _run(idx2d, partidx, weight_groups, n_workers, b, l):
    n_rows, c = idx2d.shape
    n = n_rows * c
    p, d = weight_groups.shape
    dq_n = d // 8
    bq_n = b // _C
    per_w = n // n_workers
    chunks_w = per_w // _C
    bq_w = bq_n // n_workers
    groups = _C // _LANES
    steps = bq_w * l

    mesh = plsc.VectorSubcoreMesh(core_axis_name="c", subcore_axis_name="s")

    @functools.partial(
        pl.kernel,
        mesh=mesh,
        compiler_params=pltpu.CompilerParams(use_tc_tiling_on_sc=False,
                                             needs_layout_passes=False),
        out_type=jax.ShapeDtypeStruct((l, dq_n, bq_n, 8, _C), jnp.float32),
        scratch_types=[
            pltpu.VMEM((chunks_w, _C), jnp.int32),
            pltpu.VMEM((per_w,), jnp.int32),
            pltpu.VMEM((p, d), jnp.float32),
            pltpu.VMEM((2, 64, _C), jnp.float32),
            pltpu.SemaphoreType.DMA,
            pltpu.SemaphoreType.DMA,
            pltpu.SemaphoreType.DMA,
            pltpu.SemaphoreType.DMA,
        ],
    )
    def sc_kernel(idx_hbm, part_hbm, wg_hbm, out_hbm,
                  idx_v, p_v, wg_v, tiles_v, semP, semW, semT0, semT1):
        wid = lax.axis_index("s") * 2 + lax.axis_index("c")
        row_base = wid * chunks_w

        pltpu.async_copy(wg_hbm, wg_v, semW)
        pltpu.sync_copy(idx_hbm.at[pl.ds(row_base, chunks_w)], idx_v)

        def fire_p(j, carry):
            pltpu.async_copy(part_hbm.at[idx_v.at[j]],
                             p_v.at[pl.ds(j * _C, _C)], semP)
            return carry
        lax.fori_loop(0, chunks_w, fire_p, 0)
        pltpu.make_async_copy(wg_hbm, wg_v, semW).wait()
        pltpu.make_async_copy(idx_hbm.at[pl.ds(row_base, chunks_w)], idx_v,
                              semP).wait()

        lane_l = lax.broadcasted_iota(jnp.int32, (_LANES,), 0) * l

        def drain_slot(s, sem):
            for dq in range(dq_n):
                pltpu.make_async_copy(tiles_v.at[s, pl.ds(dq * 8, 8)],
                                      out_hbm.at[0, dq, 0], sem).wait()

        def step(i, s, sem):
            bq_local = i // l
            li = i % l
            bq = wid * bq_w + bq_local

            pg = []
            for g in range(groups):
                base = (bq_local * _C + g * _LANES) * l + li
                pg.append(plsc.load_gather(p_v, [lane_l + base]))

            dcol0 = jnp.zeros((_LANES,), jnp.int32)

            @plsc.parallel_loop(0, d, 1, unroll=4, carry=dcol0)
            def dloop(dv, dcol):
                for g in range(groups):
                    vals = plsc.load_gather(wg_v, [pg[g], dcol])
                    tiles_v[s, dv, pl.ds(g * _LANES, _LANES)] = vals
                return dcol + 1

            for dq in range(dq_n):
                pltpu.async_copy(tiles_v.at[s, pl.ds(dq * 8, 8)],
                                 out_hbm.at[li, dq, bq], sem)

        def pair(t, carry):
            @pl.when(t > 0)
            def _():
                drain_slot(0, semT0)
            step(2 * t, 0, semT0)

            @pl.when(t > 0)
            def _():
                drain_slot(1, semT1)
            step(2 * t + 1, 1, semT1)
            return carry

        lax.fori_loop(0, steps // 2, pair, 0)
        drain_slot(0, semT0)
        drain_slot(1, semT1)

    return sc_kernel(idx2d, partidx, weight_groups)


def kernel(input_idx, decoder, partidx, weight_groups):
    b, l = input_idx.shape
    d = weight_groups.shape[1]
    n = b * l
    idx2d = input_idx.reshape(n // _C, _C)
    out5d = _run(idx2d, partidx, weight_groups, 32, b, l)
    return out5d.transpose(2, 4, 0, 1, 3).reshape(b, l, d)


# speedup vs baseline: 7.1105x; 4.0378x over previous
import functools

import jax
import jax.numpy as jnp
from jax import lax
from jax.experimental import pallas as pl
from jax.experimental.pallas import tpu as pltpu
from jax.experimental.pallas import tpu_sc as plsc

_C = 128
_LANES = 16


@functools.partial(jax.jit, static_argnames=("n_workers", "b", "l"))
def _run(idx2d, partidx, weight_groups, n_workers, b, l):
    n_rows, c = idx2d.shape
    n = n_rows * c
    p, d = weight_groups.shape
    dq_n = d // 8
    bq_n = b // _C
    per_w = n // n_workers
    chunks_w = per_w // _C
    bq_w = bq_n // n_workers
    groups = _C // _LANES
    steps = bq_w * l

    mesh = plsc.VectorSubcoreMesh(core_axis_name="c", subcore_axis_name="s")

    @functools.partial(
        pl.kernel,
        mesh=mesh,
        compiler_params=pltpu.CompilerParams(use_tc_tiling_on_sc=False,
                                             needs_layout_passes=False),
        out_type=jax.ShapeDtypeStruct((l, dq_n, bq_n, 8, _C), jnp.float32),
        scratch_types=[
            pltpu.VMEM((chunks_w, _C), jnp.int32),
            pltpu.VMEM((per_w,), jnp.int32),
            pltpu.VMEM((p, d), jnp.float32),
            pltpu.VMEM((d, _LANES), jnp.float32),
            pltpu.VMEM((2, 64, _C), jnp.float32),
            pltpu.SemaphoreType.DMA,
            pltpu.SemaphoreType.DMA,
            pltpu.SemaphoreType.DMA,
            pltpu.SemaphoreType.DMA,
        ],
    )
    def sc_kernel(idx_hbm, part_hbm, wg_hbm, out_hbm,
                  idx_v, p_v, wg_v, wgt_v, tiles_v, semP, semW, semT0, semT1):
        wid = lax.axis_index("s") * 2 + lax.axis_index("c")
        row_base = wid * chunks_w

        pltpu.async_copy(wg_hbm, wg_v, semW)
        pltpu.sync_copy(idx_hbm.at[pl.ds(row_base, chunks_w)], idx_v)

        def fire_p(j, carry):
            pltpu.async_copy(part_hbm.at[idx_v.at[j]],
                             p_v.at[pl.ds(j * _C, _C)], semP)
            return carry
        lax.fori_loop(0, chunks_w, fire_p, 0)
        pltpu.make_async_copy(wg_hbm, wg_v, semW).wait()
        pltpu.make_async_copy(idx_hbm.at[pl.ds(row_base, chunks_w)], idx_v,
                              semP).wait()

        lane = lax.broadcasted_iota(jnp.int32, (_LANES,), 0)
        lane_l = lane * l

        prow = jnp.where(lane < p, lane, 0)

        @plsc.parallel_loop(0, d, 1, unroll=4, carry=jnp.zeros((_LANES,),
                                                              jnp.int32))
        def build_wgt(dv, dcol):
            wgt_v[dv] = plsc.load_gather(wg_v, [prow, dcol])
            return dcol + 1

        def drain_slot(s, sem):
            for dq in range(dq_n):
                pltpu.make_async_copy(tiles_v.at[s, pl.ds(dq * 8, 8)],
                                      out_hbm.at[0, dq, 0], sem).wait()

        def step(i, s, sem):
            bq_local = i // l
            li = i % l
            bq = wid * bq_w + bq_local

            pg = []
            for g in range(groups):
                base = (bq_local * _C + g * _LANES) * l + li
                pg.append(plsc.load_gather(p_v, [lane_l + base]))

            @plsc.parallel_loop(0, d, 1, unroll=4)
            def dloop(dv):
                wg_col = wgt_v[dv]
                for g in range(groups):
                    vals = wg_col.at[pg[g]].get(mode="promise_in_bounds")
                    tiles_v[s, dv, pl.ds(g * _LANES, _LANES)] = vals

            for dq in range(dq_n):
                pltpu.async_copy(tiles_v.at[s, pl.ds(dq * 8, 8)],
                                 out_hbm.at[li, dq, bq], sem)

        def pair(t, carry):
            @pl.when(t > 0)
            def _():
                drain_slot(0, semT0)
            step(2 * t, 0, semT0)

            @pl.when(t > 0)
            def _():
                drain_slot(1, semT1)
            step(2 * t + 1, 1, semT1)
            return carry

        lax.fori_loop(0, steps // 2, pair, 0)
        drain_slot(0, semT0)
        drain_slot(1, semT1)

    return sc_kernel(idx2d, partidx, weight_groups)


def kernel(input_idx, decoder, partidx, weight_groups):
    b, l = input_idx.shape
    d = weight_groups.shape[1]
    n = b * l
    idx2d = input_idx.reshape(n // _C, _C)
    out5d = _run(idx2d, partidx, weight_groups, 32, b, l)
    return out5d.transpose(2, 4, 0, 1, 3).reshape(b, l, d)


# speedup vs baseline: 7.1127x; 1.0003x over previous
import functools

import jax
import jax.numpy as jnp
from jax import lax
from jax.experimental import pallas as pl
from jax.experimental.pallas import tpu as pltpu
from jax.experimental.pallas import tpu_sc as plsc

_C = 128
_LANES = 16


@functools.partial(jax.jit, static_argnames=("n_workers", "b", "l"))
def _run(idx2d, partidx, weight_groups, n_workers, b, l):
    n_rows, c = idx2d.shape
    n = n_rows * c
    p, d = weight_groups.shape
    dq_n = d // 8
    bq_n = b // _C
    per_w = n // n_workers
    chunks_w = per_w // _C
    bq_w = bq_n // n_workers
    groups = _C // _LANES
    steps = bq_w * l

    mesh = plsc.VectorSubcoreMesh(core_axis_name="c", subcore_axis_name="s")

    @functools.partial(
        pl.kernel,
        mesh=mesh,
        compiler_params=pltpu.CompilerParams(use_tc_tiling_on_sc=False,
                                             needs_layout_passes=False),
        out_type=jax.ShapeDtypeStruct((l, dq_n, bq_n, 8, _C), jnp.float32),
        scratch_types=[
            pltpu.VMEM((chunks_w, _C), jnp.int32),
            pltpu.VMEM((per_w,), jnp.int32),
            pltpu.VMEM((p, d), jnp.float32),
            pltpu.VMEM((d, _LANES), jnp.float32),
            pltpu.VMEM((2, 64, _C), jnp.float32),
            pltpu.SemaphoreType.DMA,
            pltpu.SemaphoreType.DMA,
            pltpu.SemaphoreType.DMA,
            pltpu.SemaphoreType.DMA,
        ],
    )
    def sc_kernel(idx_hbm, part_hbm, wg_hbm, out_hbm,
                  idx_v, p_v, wg_v, wgt_v, tiles_v, semP, semW, semT0, semT1):
        wid = lax.axis_index("s") * 2 + lax.axis_index("c")
        row_base = wid * chunks_w

        pltpu.async_copy(wg_hbm, wg_v, semW)
        pltpu.sync_copy(idx_hbm.at[pl.ds(row_base, chunks_w)], idx_v)

        def fire_p(j, carry):
            pltpu.async_copy(part_hbm.at[idx_v.at[j]],
                             p_v.at[pl.ds(j * _C, _C)], semP)
            return carry
        lax.fori_loop(0, chunks_w, fire_p, 0)
        pltpu.make_async_copy(wg_hbm, wg_v, semW).wait()
        pltpu.make_async_copy(idx_hbm.at[pl.ds(row_base, chunks_w)], idx_v,
                              semP).wait()

        lane = lax.broadcasted_iota(jnp.int32, (_LANES,), 0)
        lane_l = lane * l

        prow = jnp.where(lane < p, lane, 0)

        @plsc.parallel_loop(0, d, 1, unroll=4, carry=jnp.zeros((_LANES,),
                                                              jnp.int32))
        def build_wgt(dv, dcol):
            wgt_v[dv] = plsc.load_gather(wg_v, [prow, dcol])
            return dcol + 1

        def drain_slot(s, sem):
            for dq in range(dq_n):
                pltpu.make_async_copy(tiles_v.at[s, pl.ds(dq * 8, 8)],
                                      out_hbm.at[0, dq, 0], sem).wait()

        def step(i, s, sem):
            bq_local = i // l
            li = i % l
            bq = wid * bq_w + bq_local

            pg = []
            for g in range(groups):
                base = (bq_local * _C + g * _LANES) * l + li
                pg.append(plsc.load_gather(p_v, [lane_l + base]))

            @plsc.parallel_loop(0, d, 1, unroll=8)
            def dloop(dv):
                wg_col = wgt_v[dv]
                for g in range(groups):
                    vals = wg_col.at[pg[g]].get(mode="promise_in_bounds")
                    tiles_v[s, dv, pl.ds(g * _LANES, _LANES)] = vals

            for dq in range(dq_n):
                pltpu.async_copy(tiles_v.at[s, pl.ds(dq * 8, 8)],
                                 out_hbm.at[li, dq, bq], sem)

        def pair(t, carry):
            @pl.when(t > 0)
            def _():
                drain_slot(0, semT0)
            step(2 * t, 0, semT0)

            @pl.when(t > 0)
            def _():
                drain_slot(1, semT1)
            step(2 * t + 1, 1, semT1)
            return carry

        lax.fori_loop(0, steps // 2, pair, 0)
        drain_slot(0, semT0)
        drain_slot(1, semT1)

    return sc_kernel(idx2d, partidx, weight_groups)


def kernel(input_idx, decoder, partidx, weight_groups):
    b, l = input_idx.shape
    d = weight_groups.shape[1]
    n = b * l
    idx2d = input_idx.reshape(n // _C, _C)
    out5d = _run(idx2d, partidx, weight_groups, 32, b, l)
    return out5d.transpose(2, 4, 0, 1, 3).reshape(b, l, d)


# speedup vs baseline: 7.2414x; 1.0181x over previous
import functools

import jax
import jax.numpy as jnp
from jax import lax
from jax.experimental import pallas as pl
from jax.experimental.pallas import tpu as pltpu
from jax.experimental.pallas import tpu_sc as plsc

_C = 128
_LANES = 16


@functools.partial(jax.jit, static_argnames=("n_workers", "b", "l"))
def _run(idx2d, partidx, weight_groups, n_workers, b, l):
    n_rows, c = idx2d.shape
    n = n_rows * c
    p, d = weight_groups.shape
    dq_n = d // 8
    bq_n = b // _C
    per_w = n // n_workers
    chunks_w = per_w // _C
    bq_w = bq_n // n_workers
    groups = _C // _LANES
    steps = bq_w * l

    mesh = plsc.VectorSubcoreMesh(core_axis_name="c", subcore_axis_name="s")

    @functools.partial(
        pl.kernel,
        mesh=mesh,
        compiler_params=pltpu.CompilerParams(use_tc_tiling_on_sc=False,
                                             needs_layout_passes=False),
        out_type=jax.ShapeDtypeStruct((l, dq_n, bq_n, 8, _C), jnp.float32),
        scratch_types=[
            pltpu.VMEM((chunks_w, _C), jnp.int32),
            pltpu.VMEM((per_w,), jnp.int32),
            pltpu.VMEM((p, d), jnp.float32),
            pltpu.VMEM((d, _LANES), jnp.float32),
            pltpu.VMEM((2, 64, _C), jnp.float32),
            pltpu.SemaphoreType.DMA((4,)),
            pltpu.SemaphoreType.DMA,
            pltpu.SemaphoreType.DMA,
            pltpu.SemaphoreType.DMA,
        ],
    )
    def sc_kernel(idx_hbm, part_hbm, wg_hbm, out_hbm,
                  idx_v, p_v, wg_v, wgt_v, tiles_v, semP, semW, semT0, semT1):
        wid = lax.axis_index("s") * 2 + lax.axis_index("c")
        row_base = wid * chunks_w

        pltpu.async_copy(wg_hbm, wg_v, semW)
        pltpu.sync_copy(idx_hbm.at[pl.ds(row_base, chunks_w)], idx_v)

        cpb = chunks_w // bq_w
        for blk in range(bq_w):
            def fire_p(j, carry, blk=blk):
                pltpu.async_copy(part_hbm.at[idx_v.at[j]],
                                 p_v.at[pl.ds(j * _C, _C)], semP.at[blk])
                return carry
            lax.fori_loop(blk * cpb, (blk + 1) * cpb, fire_p, 0)
        pltpu.make_async_copy(wg_hbm, wg_v, semW).wait()

        lane = lax.broadcasted_iota(jnp.int32, (_LANES,), 0)
        lane_l = lane * l

        prow = jnp.where(lane < p, lane, 0)

        @plsc.parallel_loop(0, d, 1, unroll=4, carry=jnp.zeros((_LANES,),
                                                              jnp.int32))
        def build_wgt(dv, dcol):
            wgt_v[dv] = plsc.load_gather(wg_v, [prow, dcol])
            return dcol + 1

        def drain_slot(s, sem):
            for dq in range(dq_n):
                pltpu.make_async_copy(tiles_v.at[s, pl.ds(dq * 8, 8)],
                                      out_hbm.at[0, dq, 0], sem).wait()

        def step(i, s, sem):
            bq_local = i // l
            li = i % l
            bq = wid * bq_w + bq_local

            pg = []
            for g in range(groups):
                base = (bq_local * _C + g * _LANES) * l + li
                pg.append(plsc.load_gather(p_v, [lane_l + base]))

            @plsc.parallel_loop(0, d, 1, unroll=4)
            def dloop(dv):
                wg_col = wgt_v[dv]
                for g in range(groups):
                    vals = wg_col.at[pg[g]].get(mode="promise_in_bounds")
                    tiles_v[s, dv, pl.ds(g * _LANES, _LANES)] = vals

            for dq in range(dq_n):
                pltpu.async_copy(tiles_v.at[s, pl.ds(dq * 8, 8)],
                                 out_hbm.at[li, dq, bq], sem)

        for blk in range(bq_w):
            pltpu.make_async_copy(idx_hbm.at[pl.ds(row_base, cpb)],
                                  idx_v.at[pl.ds(0, cpb)], semP.at[blk]).wait()

            def pair(t, carry, blk=blk):
                i0 = blk * l + 2 * t
                if blk == 0:
                    @pl.when(t > 0)
                    def _():
                        drain_slot(0, semT0)
                else:
                    drain_slot(0, semT0)
                step(i0, 0, semT0)

                if blk == 0:
                    @pl.when(t > 0)
                    def _():
                        drain_slot(1, semT1)
                else:
                    drain_slot(1, semT1)
                step(i0 + 1, 1, semT1)
                return carry

            lax.fori_loop(0, l // 2, pair, 0)
        drain_slot(0, semT0)
        drain_slot(1, semT1)

    return sc_kernel(idx2d, partidx, weight_groups)


def kernel(input_idx, decoder, partidx, weight_groups):
    b, l = input_idx.shape
    d = weight_groups.shape[1]
    n = b * l
    idx2d = input_idx.reshape(n // _C, _C)
    out5d = _run(idx2d, partidx, weight_groups, 32, b, l)
    return out5d.transpose(2, 4, 0, 1, 3).reshape(b, l, d)
